# baseline (speedup 1.0000x reference)
import functools

import jax
import jax.numpy as jnp
from jax import lax
from jax.experimental import pallas as pl
from jax.experimental.pallas import tpu as pltpu
from jax.experimental.pallas import tpu_sc as plsc

R = 16
N = 2048
B = 4096
X = 8192
NB = 8
E = 2 ** NB
L = 16
HALF = N // 2
NG = 8
GSZ = HALF // NG
CH = GSZ // L

_mesh = plsc.VectorSubcoreMesh(core_axis_name="c", subcore_axis_name="s")


@functools.partial(
    pl.kernel,
    out_type=jax.ShapeDtypeStruct((N,), jnp.int32),
    mesh=_mesh,
    compiler_params=pltpu.CompilerParams(
        needs_layout_passes=False, use_tc_tiling_on_sc=True),
    scratch_types=[
        pltpu.VMEM((X,), jnp.int32),
        pltpu.VMEM((32, 128), jnp.int32),
        pltpu.VMEM((2, GSZ, NB), jnp.int32),
        pltpu.VMEM((2, GSZ, E), jnp.float32),
        pltpu.VMEM((16, 128), jnp.int32),
        pltpu.VMEM((16,), jnp.int32),
        pltpu.VMEM((16, 128), jnp.int32),
        pltpu.VMEM((GSZ,), jnp.int32),
        pltpu.VMEM_SHARED((16, 128), jnp.int32),
        pltpu.SemaphoreType.DMA,
        pltpu.SemaphoreType.DMA,
        pltpu.SemaphoreType.DMA,
    ],
)
def _ensemble_ram_sc(x_hbm, proj_hbm, conn_hbm, mem_hbm, out_hbm,
                     x_v, proj_v, conn_v, rows_v, bits_v,
                     rowidx_v, red_v, outst_v, shared,
                     sem_in, sem_mem, sem_conn):
    cid = lax.axis_index("c")
    sid = lax.axis_index("s")
    lane = lax.iota(jnp.int32, L)
    n0 = cid * HALF

    for g in range(2):
        pltpu.async_copy(
            mem_hbm.at[sid, pl.ds(n0 + g * GSZ, GSZ)], rows_v.at[g], sem_mem)
        pltpu.async_copy(
            conn_hbm.at[sid, pl.ds(n0 + g * GSZ, GSZ)], conn_v.at[g], sem_conn)
    in_copies = [
        pltpu.async_copy(x_hbm, x_v, sem_in),
        pltpu.async_copy(proj_hbm.at[sid], proj_v, sem_in),
    ]
    for c in in_copies:
        c.wait()

    zero = jnp.zeros((L,), jnp.int32)
    for t in range(8, 16):
        for k in range(8):
            bits_v[t, pl.ds(k * L, L)] = zero

    def group_body(gi, _):
        for half in range(2):
            g = gi * 2 + half
            pltpu.make_async_copy(
                mem_hbm.at[sid, pl.ds(n0, GSZ)], rows_v.at[half], sem_mem
            ).wait()
            pltpu.make_async_copy(
                conn_hbm.at[sid, pl.ds(n0, GSZ)], conn_v.at[half], sem_conn
            ).wait()
            for i in range(CH):
                rowvec = i * L + lane
                addr = jnp.zeros((L,), jnp.int32)
                for b in range(NB):
                    c = plsc.load_gather(
                        conn_v.at[half], [rowvec, jnp.full((L,), b, jnp.int32)])
                    w = plsc.load_gather(
                        proj_v, [jnp.right_shift(c, 7), jnp.bitwise_and(c, 127)])
                    bit = plsc.load_gather(x_v, [w])
                    addr = addr + bit * (1 << b)
                val = plsc.load_gather(rows_v.at[half], [rowvec, addr])
                bits_v[g, pl.ds(i * L, L)] = (
                    jnp.where(val > 0.5, 1, 0).astype(jnp.int32))

            @pl.when(g < NG - 2)
            def _():
                pltpu.async_copy(
                    mem_hbm.at[sid, pl.ds(n0 + (g + 2) * GSZ, GSZ)],
                    rows_v.at[half], sem_mem)
                pltpu.async_copy(
                    conn_hbm.at[sid, pl.ds(n0 + (g + 2) * GSZ, GSZ)],
                    conn_v.at[half], sem_conn)
        return _

    lax.fori_loop(0, NG // 2, group_body, None)

    rowidx_v[...] = lane

    @pl.when(sid == 0)
    def _():
        pltpu.sync_copy(bits_v, shared)

    plsc.subcore_barrier()

    @pl.when(sid != 0)
    def _():
        pltpu.sync_copy(bits_v, shared.at[rowidx_v], add=True)

    plsc.subcore_barrier()

    @pl.when(sid < NG)
    def _():
        pltpu.sync_copy(shared, red_v)
        for k in range(CH):
            acc = red_v[sid, pl.ds(k * L, L)]
            outst_v[pl.ds(k * L, L)] = jnp.where(acc > R // 2, 1, 0).astype(jnp.int32)
        pltpu.sync_copy(outst_v, out_hbm.at[pl.ds(cid * HALF + sid * GSZ, GSZ)])


def kernel(x, projections, conn, memory):
    proj3 = projections.reshape(R, 32, 128)
    out = _ensemble_ram_sc(x, proj3, conn, memory)
    return out.astype(jnp.uint8)
